# baseline (speedup 1.0000x reference)
import functools

import jax
import jax.numpy as jnp
from jax import lax
from jax.experimental import pallas as pl
from jax.experimental.pallas import tpu as pltpu
from jax.experimental.pallas import tpu_sc as plsc

_N = 1000000
_NPAD = 1000064
_BATCH = 16384
_K = 32
_L = 16
_NC, _NS = 2, 16
_NW = _NC * _NS
_BPW = _BATCH // _NW
_BLKS = _BPW // _L

_S0 = 770688
_SSC = _NPAD - _S0
_CH = 256
_CPW = 28
_COLSPW = _CH * _CPW

_BLK = 32768
_G = -(-_S0 // _BLK)


def _matvec_body(wt_ref, ht_ref, wu_ref, wv_ref, au_ref, av_ref):
    wu = wu_ref[...][:, 0:1]
    wv = wv_ref[...][:, 0:1]
    au_ref[...] = jnp.sum(wt_ref[...] * wu, axis=0)
    av_ref[...] = jnp.sum(ht_ref[...] * wv, axis=0)


_matvec = pl.pallas_call(
    _matvec_body,
    grid=(_G,),
    in_specs=[
        pl.BlockSpec((_K, _BLK), lambda i: (0, i)),
        pl.BlockSpec((_K, _BLK), lambda i: (0, i)),
        pl.BlockSpec((_K, 128), lambda i: (0, 0)),
        pl.BlockSpec((_K, 128), lambda i: (0, 0)),
    ],
    out_specs=[
        pl.BlockSpec((_BLK,), lambda i: (i,)),
        pl.BlockSpec((_BLK,), lambda i: (i,)),
    ],
    out_shape=[jax.ShapeDtypeStruct((_S0,), jnp.float32)] * 2,
)

_mesh = plsc.VectorSubcoreMesh(core_axis_name="c", subcore_axis_name="s")


@functools.partial(
    pl.kernel,
    mesh=_mesh,
    out_type=[jax.ShapeDtypeStruct((_SSC,), jnp.float32)] * 2,
    scratch_types=[
        pltpu.VMEM((_K, _CH), jnp.float32),
        pltpu.VMEM((_K, _CH), jnp.float32),
        pltpu.VMEM((_K, _CH), jnp.float32),
        pltpu.VMEM((_K, _CH), jnp.float32),
        pltpu.VMEM((2 * _K, _L), jnp.float32),
        pltpu.VMEM((_COLSPW,), jnp.float32),
        pltpu.VMEM((_COLSPW,), jnp.float32),
        pltpu.SemaphoreType.DMA,
        pltpu.SemaphoreType.DMA,
        pltpu.SemaphoreType.DMA,
        pltpu.SemaphoreType.DMA,
    ],
)
def _tail_mv(wt_hbm, ht_hbm, wb_hbm, au_hbm, av_hbm,
             bw0, bw1, bh0, bh1, wb_v, outu_v, outv_v, sw0, sw1, sh0, sh1):
    wid = lax.axis_index("s") * _NC + lax.axis_index("c")
    base_col = _S0 + wid * _COLSPW
    pltpu.sync_copy(wb_hbm, wb_v)
    bufw, bufh = (bw0, bw1), (bh0, bh1)
    semw, semh = (sw0, sw1), (sh0, sh1)

    def chunk_src(ci, tab_hbm):
        c0 = pl.multiple_of(base_col + ci * _CH, 128)
        return tab_hbm.at[:, pl.ds(c0, _CH)]

    def issue(ci, b):
        pltpu.async_copy(chunk_src(ci, wt_hbm), bufw[b], semw[b])
        pltpu.async_copy(chunk_src(ci, ht_hbm), bufh[b], semh[b])

    def drain(ci, b):
        pltpu.make_async_copy(chunk_src(ci, wt_hbm), bufw[b], semw[b]).wait()
        pltpu.make_async_copy(chunk_src(ci, ht_hbm), bufh[b], semh[b]).wait()

    def comp_one(buf, wrow0, out_v, ci):
        for g in range(_CH // 128):
            accs = []
            w0 = wb_v[wrow0, pl.ds(0, _L)]
            for bb in range(8):
                accs.append(buf[0, pl.ds(g * 128 + bb * _L, _L)] * w0)
            for k in range(1, _K):
                wk = wb_v[wrow0 + k, pl.ds(0, _L)]
                for bb in range(8):
                    accs[bb] = accs[bb] + buf[k, pl.ds(g * 128 + bb * _L, _L)] * wk
            for bb in range(8):
                out_v[pl.ds(ci * _CH + g * 128 + bb * _L, _L)] = accs[bb]

    issue(0, 0)

    def pair(cp, carry):
        ci0 = cp * 2
        issue(ci0 + 1, 1)
        drain(ci0, 0)
        comp_one(bufw[0], 0, outu_v, ci0)
        comp_one(bufh[0], _K, outv_v, ci0)

        @pl.when(cp < _CPW // 2 - 1)
        def _():
            issue(ci0 + 2, 0)

        drain(ci0 + 1, 1)
        comp_one(bufw[1], 0, outu_v, ci0 + 1)
        comp_one(bufh[1], _K, outv_v, ci0 + 1)
        return carry

    lax.fori_loop(0, _CPW // 2, pair, 0)
    pltpu.sync_copy(outu_v, au_hbm.at[pl.ds(wid * _COLSPW, _COLSPW)])
    pltpu.sync_copy(outv_v, av_hbm.at[pl.ds(wid * _COLSPW, _COLSPW)])


@functools.partial(
    pl.kernel,
    mesh=_mesh,
    out_type=jax.ShapeDtypeStruct((_BATCH,), jnp.float32),
    scratch_types=[
        pltpu.VMEM((_BPW,), jnp.int32),
        pltpu.VMEM((_BPW,), jnp.int32),
        pltpu.VMEM((_BPW,), jnp.int32),
        pltpu.VMEM((_BPW,), jnp.int32),
        pltpu.VMEM((_BPW,), jnp.int32),
        pltpu.VMEM((_BPW,), jnp.int32),
        pltpu.VMEM((_BPW,), jnp.float32),
        pltpu.VMEM((_BPW,), jnp.float32),
        pltpu.VMEM((_BPW,), jnp.float32),
        pltpu.VMEM((_BPW,), jnp.float32),
        pltpu.VMEM((_L,), jnp.float32),
        pltpu.VMEM((_BPW,), jnp.float32),
        pltpu.SemaphoreType.DMA,
        pltpu.SemaphoreType.DMA,
        pltpu.SemaphoreType.DMA,
        pltpu.SemaphoreType.DMA,
    ],
)
def _gather_sig(uidx_hbm, vidx_hbm, autc_hbm, avtc_hbm, ausc_hbm, avsc_hbm,
                wb_hbm, out_hbm,
                uidx_v, vidx_v, iutc_v, iusc_v, ivtc_v, ivsc_v,
                gutc_v, gusc_v, gvtc_v, gvsc_v, wb_v, out_v,
                s0, s1, s2, s3):
    wid = lax.axis_index("s") * _NC + lax.axis_index("c")
    base = wid * _BPW
    pltpu.sync_copy(uidx_hbm.at[pl.ds(base, _BPW)], uidx_v)
    pltpu.sync_copy(vidx_hbm.at[pl.ds(base, _BPW)], vidx_v)
    pltpu.sync_copy(wb_hbm, wb_v)
    for blk in range(_BLKS):
        sl = pl.ds(blk * _L, _L)
        iu = uidx_v[sl]
        iv = vidx_v[sl]
        iutc_v[sl] = jnp.minimum(iu, _S0 - 1)
        iusc_v[sl] = jnp.maximum(iu - _S0, 0)
        ivtc_v[sl] = jnp.minimum(iv, _S0 - 1)
        ivsc_v[sl] = jnp.maximum(iv - _S0, 0)
    c0 = pltpu.async_copy(autc_hbm.at[iutc_v], gutc_v, s0)
    c1 = pltpu.async_copy(ausc_hbm.at[iusc_v], gusc_v, s1)
    c2 = pltpu.async_copy(avtc_hbm.at[ivtc_v], gvtc_v, s2)
    c3 = pltpu.async_copy(avsc_hbm.at[ivsc_v], gvsc_v, s3)
    c0.wait()
    c1.wait()
    c2.wait()
    c3.wait()
    bias = wb_v[...]
    for blk in range(_BLKS):
        sl = pl.ds(blk * _L, _L)
        iu = uidx_v[sl]
        iv = vidx_v[sl]
        hu = jnp.where(iu >= _S0, gusc_v[sl], gutc_v[sl])
        hv = jnp.where(iv >= _S0, gvsc_v[sl], gvtc_v[sl])
        h = hu + hv + bias
        out_v[sl] = 1.0 / (1.0 + jnp.exp(-h))
    pltpu.sync_copy(out_v, out_hbm.at[pl.ds(base, _BPW)])


def kernel(x, W, H, lin_w, lin_b):
    uidx = x[:, 0]
    vidx = x[:, 1]
    wu_b = jnp.broadcast_to(lin_w[0:_K], (_K, 128))
    wv_b = jnp.broadcast_to(lin_w[_K:2 * _K], (_K, 128))
    wb64 = jnp.broadcast_to(lin_w, (2 * _K, _L))
    au_tc, av_tc = _matvec(W.T, H.T, wu_b, wv_b)
    au_sc, av_sc = _tail_mv(W.T, H.T, wb64)
    bias16 = jnp.broadcast_to(lin_b, (_L,))
    return _gather_sig(uidx, vidx, au_tc, av_tc, au_sc, av_sc, bias16)


# speedup vs baseline: 1.5774x; 1.5774x over previous
import functools

import jax
import jax.numpy as jnp
from jax import lax
from jax.experimental import pallas as pl
from jax.experimental.pallas import tpu as pltpu
from jax.experimental.pallas import tpu_sc as plsc

_N = 1000000
_NPAD = 1000064
_BATCH = 16384
_K = 32
_L = 16
_NC, _NS = 2, 16
_NW = _NC * _NS
_BPW = _BATCH // _NW
_BLKS = _BPW // _L

_S0 = 803456
_SSC = _NPAD - _S0
_CH = 256
_CPW = 24
_COLSPW = _CH * _CPW

_BLK = 32768
_G = -(-_S0 // _BLK)


def _matvec_body(wt_ref, ht_ref, wu_ref, wv_ref, au_ref, av_ref):
    wu = wu_ref[...][:, 0:1]
    wv = wv_ref[...][:, 0:1]
    au_ref[...] = jnp.sum(wt_ref[...] * wu, axis=0)
    av_ref[...] = jnp.sum(ht_ref[...] * wv, axis=0)


_matvec = pl.pallas_call(
    _matvec_body,
    grid=(_G,),
    in_specs=[
        pl.BlockSpec((_K, _BLK), lambda i: (0, i)),
        pl.BlockSpec((_K, _BLK), lambda i: (0, i)),
        pl.BlockSpec((_K, 128), lambda i: (0, 0)),
        pl.BlockSpec((_K, 128), lambda i: (0, 0)),
    ],
    out_specs=[
        pl.BlockSpec((_BLK,), lambda i: (i,)),
        pl.BlockSpec((_BLK,), lambda i: (i,)),
    ],
    out_shape=[jax.ShapeDtypeStruct((_S0,), jnp.float32)] * 2,
)

_mesh = plsc.VectorSubcoreMesh(core_axis_name="c", subcore_axis_name="s")


@functools.partial(
    pl.kernel,
    mesh=_mesh,
    out_type=[jax.ShapeDtypeStruct((_SSC,), jnp.float32)] * 2,
    scratch_types=[
        pltpu.VMEM((_K, _CH), jnp.float32),
        pltpu.VMEM((_K, _CH), jnp.float32),
        pltpu.VMEM((_K, _CH), jnp.float32),
        pltpu.VMEM((_K, _CH), jnp.float32),
        pltpu.VMEM((2 * _K, _L), jnp.float32),
        pltpu.VMEM((_COLSPW,), jnp.float32),
        pltpu.VMEM((_COLSPW,), jnp.float32),
        pltpu.SemaphoreType.DMA,
        pltpu.SemaphoreType.DMA,
        pltpu.SemaphoreType.DMA,
        pltpu.SemaphoreType.DMA,
    ],
)
def _tail_mv(wt_hbm, ht_hbm, wb_hbm, au_hbm, av_hbm,
             bw0, bw1, bh0, bh1, wb_v, outu_v, outv_v, sw0, sw1, sh0, sh1):
    wid = lax.axis_index("s") * _NC + lax.axis_index("c")
    base_col = _S0 + wid * _COLSPW
    pltpu.sync_copy(wb_hbm, wb_v)
    bufw, bufh = (bw0, bw1), (bh0, bh1)
    semw, semh = (sw0, sw1), (sh0, sh1)

    def chunk_src(ci, tab_hbm):
        c0 = pl.multiple_of(base_col + ci * _CH, 128)
        return tab_hbm.at[:, pl.ds(c0, _CH)]

    def issue(ci, b):
        pltpu.async_copy(chunk_src(ci, wt_hbm), bufw[b], semw[b])
        pltpu.async_copy(chunk_src(ci, ht_hbm), bufh[b], semh[b])

    def drain(ci, b):
        pltpu.make_async_copy(chunk_src(ci, wt_hbm), bufw[b], semw[b]).wait()
        pltpu.make_async_copy(chunk_src(ci, ht_hbm), bufh[b], semh[b]).wait()

    def comp_one(buf, wrow0, out_v, ci):
        for g in range(_CH // 128):
            accs = []
            w0 = wb_v[wrow0, pl.ds(0, _L)]
            for bb in range(8):
                accs.append(buf[0, pl.ds(g * 128 + bb * _L, _L)] * w0)
            for k in range(1, _K):
                wk = wb_v[wrow0 + k, pl.ds(0, _L)]
                for bb in range(8):
                    accs[bb] = accs[bb] + buf[k, pl.ds(g * 128 + bb * _L, _L)] * wk
            for bb in range(8):
                out_v[pl.ds(ci * _CH + g * 128 + bb * _L, _L)] = accs[bb]

    issue(0, 0)

    def pair(cp, carry):
        ci0 = cp * 2
        issue(ci0 + 1, 1)
        drain(ci0, 0)
        comp_one(bufw[0], 0, outu_v, ci0)
        comp_one(bufh[0], _K, outv_v, ci0)

        @pl.when(cp < _CPW // 2 - 1)
        def _():
            issue(ci0 + 2, 0)

        drain(ci0 + 1, 1)
        comp_one(bufw[1], 0, outu_v, ci0 + 1)
        comp_one(bufh[1], _K, outv_v, ci0 + 1)
        return carry

    lax.fori_loop(0, _CPW // 2, pair, 0)
    pltpu.sync_copy(outu_v, au_hbm.at[pl.ds(wid * _COLSPW, _COLSPW)])
    pltpu.sync_copy(outv_v, av_hbm.at[pl.ds(wid * _COLSPW, _COLSPW)])


@functools.partial(
    pl.kernel,
    mesh=_mesh,
    out_type=jax.ShapeDtypeStruct((_BATCH,), jnp.float32),
    scratch_types=[
        pltpu.VMEM((_BPW,), jnp.int32),
        pltpu.VMEM((_BPW,), jnp.int32),
        pltpu.VMEM((_BPW,), jnp.int32),
        pltpu.VMEM((_BPW,), jnp.int32),
        pltpu.VMEM((_BPW,), jnp.int32),
        pltpu.VMEM((_BPW,), jnp.int32),
        pltpu.VMEM((_BPW,), jnp.float32),
        pltpu.VMEM((_BPW,), jnp.float32),
        pltpu.VMEM((_BPW,), jnp.float32),
        pltpu.VMEM((_BPW,), jnp.float32),
        pltpu.VMEM((_L,), jnp.float32),
        pltpu.VMEM((_BPW,), jnp.float32),
        pltpu.SemaphoreType.DMA,
        pltpu.SemaphoreType.DMA,
        pltpu.SemaphoreType.DMA,
        pltpu.SemaphoreType.DMA,
    ],
)
def _gather_sig(uidx_hbm, vidx_hbm, autc_hbm, avtc_hbm, ausc_hbm, avsc_hbm,
                wb_hbm, out_hbm,
                uidx_v, vidx_v, iutc_v, iusc_v, ivtc_v, ivsc_v,
                gutc_v, gusc_v, gvtc_v, gvsc_v, wb_v, out_v,
                s0, s1, s2, s3):
    wid = lax.axis_index("s") * _NC + lax.axis_index("c")
    base = wid * _BPW
    pltpu.sync_copy(uidx_hbm.at[pl.ds(base, _BPW)], uidx_v)
    pltpu.sync_copy(vidx_hbm.at[pl.ds(base, _BPW)], vidx_v)
    pltpu.sync_copy(wb_hbm, wb_v)
    for blk in range(_BLKS):
        sl = pl.ds(blk * _L, _L)
        iu = uidx_v[sl]
        iv = vidx_v[sl]
        iutc_v[sl] = jnp.where(iu < _S0, iu, iu >> 2)
        iusc_v[sl] = jnp.where(iu >= _S0, iu - _S0, iu >> 3)
        ivtc_v[sl] = jnp.where(iv < _S0, iv, iv >> 2)
        ivsc_v[sl] = jnp.where(iv >= _S0, iv - _S0, iv >> 3)
    c0 = pltpu.async_copy(autc_hbm.at[iutc_v], gutc_v, s0)
    c1 = pltpu.async_copy(ausc_hbm.at[iusc_v], gusc_v, s1)
    c2 = pltpu.async_copy(avtc_hbm.at[ivtc_v], gvtc_v, s2)
    c3 = pltpu.async_copy(avsc_hbm.at[ivsc_v], gvsc_v, s3)
    c0.wait()
    c1.wait()
    c2.wait()
    c3.wait()
    bias = wb_v[...]
    for blk in range(_BLKS):
        sl = pl.ds(blk * _L, _L)
        iu = uidx_v[sl]
        iv = vidx_v[sl]
        hu = jnp.where(iu >= _S0, gusc_v[sl], gutc_v[sl])
        hv = jnp.where(iv >= _S0, gvsc_v[sl], gvtc_v[sl])
        h = hu + hv + bias
        out_v[sl] = 1.0 / (1.0 + jnp.exp(-h))
    pltpu.sync_copy(out_v, out_hbm.at[pl.ds(base, _BPW)])


def kernel(x, W, H, lin_w, lin_b):
    uidx = x[:, 0]
    vidx = x[:, 1]
    wu_b = jnp.broadcast_to(lin_w[0:_K], (_K, 128))
    wv_b = jnp.broadcast_to(lin_w[_K:2 * _K], (_K, 128))
    wb64 = jnp.broadcast_to(lin_w, (2 * _K, _L))
    au_tc, av_tc = _matvec(W.T, H.T, wu_b, wv_b)
    au_sc, av_sc = _tail_mv(W.T, H.T, wb64)
    bias16 = jnp.broadcast_to(lin_b, (_L,))
    return _gather_sig(uidx, vidx, au_tc, av_tc, au_sc, av_sc, bias16)


# speedup vs baseline: 1.6518x; 1.0472x over previous
import functools

import jax
import jax.numpy as jnp
from jax import lax
from jax.experimental import pallas as pl
from jax.experimental.pallas import tpu as pltpu
from jax.experimental.pallas import tpu_sc as plsc

_N = 1000000
_BATCH = 16384
_K = 32
_L = 16
_NC, _NS = 2, 16
_NW = _NC * _NS
_BPW = _BATCH // _NW
_BLKS = _BPW // _L

_BLK = 32768
_G = -(-_N // _BLK)


def _matvec_body(wt_ref, ht_ref, wu_ref, wv_ref, au_ref, av_ref):
    wu = wu_ref[...][:, 0:1]
    wv = wv_ref[...][:, 0:1]
    au_ref[...] = jnp.sum(wt_ref[...] * wu, axis=0)
    av_ref[...] = jnp.sum(ht_ref[...] * wv, axis=0)


_matvec = pl.pallas_call(
    _matvec_body,
    grid=(_G,),
    in_specs=[
        pl.BlockSpec((_K, _BLK), lambda i: (0, i)),
        pl.BlockSpec((_K, _BLK), lambda i: (0, i)),
        pl.BlockSpec((_K, 128), lambda i: (0, 0)),
        pl.BlockSpec((_K, 128), lambda i: (0, 0)),
    ],
    out_specs=[
        pl.BlockSpec((_BLK,), lambda i: (i,)),
        pl.BlockSpec((_BLK,), lambda i: (i,)),
    ],
    out_shape=[jax.ShapeDtypeStruct((_N,), jnp.float32)] * 2,
)

_mesh = plsc.VectorSubcoreMesh(core_axis_name="c", subcore_axis_name="s")


@functools.partial(
    pl.kernel,
    mesh=_mesh,
    out_type=jax.ShapeDtypeStruct((_BATCH,), jnp.float32),
    scratch_types=[
        pltpu.VMEM((2, _BPW), jnp.int32),
        pltpu.VMEM((_BPW,), jnp.int32),
        pltpu.VMEM((_BPW,), jnp.int32),
        pltpu.VMEM((_BPW,), jnp.float32),
        pltpu.VMEM((_BPW,), jnp.float32),
        pltpu.VMEM((_L,), jnp.float32),
        pltpu.VMEM((_BPW,), jnp.float32),
        pltpu.SemaphoreType.DMA,
        pltpu.SemaphoreType.DMA,
    ],
)
def _gather_sig(xt_hbm, au_hbm, av_hbm, wb_hbm, out_hbm,
                x2_v, uidx_v, vidx_v, au_v, av_v, wb_v, out_v, sem_u, sem_v):
    wid = lax.axis_index("s") * _NC + lax.axis_index("c")
    base = pl.multiple_of(wid * _BPW, 128)
    pltpu.sync_copy(xt_hbm.at[:, pl.ds(base, _BPW)], x2_v)
    pltpu.sync_copy(wb_hbm, wb_v)
    for blk in range(_BLKS):
        sl = pl.ds(blk * _L, _L)
        uidx_v[sl] = x2_v[0, sl]
        vidx_v[sl] = x2_v[1, sl]
    cu = pltpu.async_copy(au_hbm.at[uidx_v], au_v, sem_u)
    cv = pltpu.async_copy(av_hbm.at[vidx_v], av_v, sem_v)
    cu.wait()
    cv.wait()
    bias = wb_v[...]
    for blk in range(_BLKS):
        h = au_v[pl.ds(blk * _L, _L)] + av_v[pl.ds(blk * _L, _L)] + bias
        out_v[pl.ds(blk * _L, _L)] = 1.0 / (1.0 + jnp.exp(-h))
    pltpu.sync_copy(out_v, out_hbm.at[pl.ds(base, _BPW)])


def kernel(x, W, H, lin_w, lin_b):
    wu_b = jnp.broadcast_to(lin_w[0:_K], (_K, 128))
    wv_b = jnp.broadcast_to(lin_w[_K:2 * _K], (_K, 128))
    au, av = _matvec(W.T, H.T, wu_b, wv_b)
    bias16 = jnp.broadcast_to(lin_b, (_L,))
    return _gather_sig(x.T, au, av, bias16)
